# baseline (speedup 1.0000x reference)
import functools

import jax
import jax.numpy as jnp
from jax import lax
from jax.experimental import pallas as pl
from jax.experimental.pallas import tpu as pltpu
from jax.experimental.pallas import tpu_sc as plsc

F32 = jnp.float32
BF16 = jnp.bfloat16
L = 16
NC = 2
NS = 16
NW = NC * NS
CH = 1024
KB = CH // 128
CHG = 128


def _gelu(x):
    return x * 0.5 * (1.0 + lax.erf(x * 0.7071067811865476))


def _qdot(a, b):
    return jnp.dot(a.astype(BF16), b.astype(BF16), preferred_element_type=F32)


def _fdot(a, b):
    return jnp.dot(a, b, preferred_element_type=F32,
                   precision=lax.Precision.HIGHEST)


def _mlp4_body(x_ref, w1, b1, w2, b2, w3, b3, w4, b4, o_ref):
    h = _gelu(_qdot(x_ref[:], w1[:]) + b1[:])
    h = _gelu(_qdot(h, w2[:]) + b2[:])
    h = _gelu(_qdot(h, w3[:]) + b3[:])
    o_ref[:] = _qdot(h, w4[:]) + b4[:]


def _bn_body(v, g_ref, bta_ref):
    mean = jnp.mean(v, axis=0, keepdims=True)
    var = jnp.mean((v - mean) ** 2, axis=0, keepdims=True)
    return (v - mean) / jnp.sqrt(var + 1e-5) * g_ref[:] + bta_ref[:]


def _bn0_body(v_ref, g_ref, bta_ref, vbn_ref):
    vbn_ref[:] = _bn_body(v_ref[:], g_ref, bta_ref)


def _update(n_rows, v_ref, s_ref, d_ref, w4t_ref, b4_ref):
    s = s_ref[0, :n_rows, :] + s_ref[1, :n_rows, :]
    deg = d_ref[0, :n_rows, 0:1] + d_ref[1, :n_rows, 0:1]
    w4q = w4t_ref[:].astype(BF16).astype(F32)
    return v_ref[:] + _fdot(s, w4q) + deg * b4_ref[:]


def _upd_bn_body(n_rows, v_ref, s_ref, d_ref, w4t_ref, b4_ref,
                 g_ref, bta_ref, vbn_ref):
    vbn_ref[:] = _bn_body(_update(n_rows, v_ref, s_ref, d_ref, w4t_ref, b4_ref),
                          g_ref, bta_ref)


def _upd_dec_body(n_rows, v_ref, s_ref, d_ref, w4t_ref, b4_ref,
                  dw1, db1, dw2, db2, dw3, db3, dw4, db4, o_ref):
    v = _update(n_rows, v_ref, s_ref, d_ref, w4t_ref, b4_ref)
    h = _gelu(_qdot(v, dw1[:]) + db1[:])
    h = _gelu(_qdot(h, dw2[:]) + db2[:])
    h = _gelu(_qdot(h, dw3[:]) + db3[:])
    o_ref[:] = _qdot(h, dw4[:]) + db4[:]


def _edge_enc_body(ea_ref, w1, b1, w2, b2, w3, b3, o_ref):
    h = _gelu(_qdot(ea_ref[:], w1[:]) + b1[:])
    h = _gelu(_qdot(h, w2[:]) + b2[:])
    o_ref[:] = _gelu(_qdot(h, w3[:]) + b3[:])


def _edge_enc4_body(g3_ref, w4, b4, o_ref):
    o_ref[:] = _qdot(g3_ref[:], w4[:]) + b4[:]


def _mp_l1_body(g0_ref, g1_ref, e_ref, w1, b1, o_ref):
    m = jnp.concatenate([g0_ref[0], g1_ref[0]], axis=1) + e_ref[:]
    o_ref[:] = _gelu(_qdot(m, w1[:]) + b1[:])


def _mp_l23_body(h_ref, w2, b2, w3, b3, o_ref):
    h = _gelu(_qdot(h_ref[:], w2[:]) + b2[:])
    h = _gelu(_qdot(h, w3[:]) + b3[:])
    o_ref[:] = h.astype(BF16).astype(F32)


def _make_sc_gather(ep, n_tab, d):
    dh = d // 2
    per_t = ep // NS
    chunks = per_t // CHG
    idx_rows_per_t = per_t // 128
    tab_rows_per_tile = n_tab // NS
    mesh = plsc.VectorSubcoreMesh(core_axis_name="c", subcore_axis_name="s")

    @functools.partial(
        pl.kernel,
        mesh=mesh,
        compiler_params=pltpu.CompilerParams(use_tc_tiling_on_sc=False),
        out_type=jax.ShapeDtypeStruct((NC, ep, dh), F32),
        scratch_types=[
            pltpu.VMEM((idx_rows_per_t, 128), jnp.int32),
            pltpu.VMEM((idx_rows_per_t, 128), jnp.int32),
            pltpu.VMEM((CHG, dh), F32),
            pltpu.VMEM((CHG, dh), F32),
            pltpu.VMEM((CHG, dh), F32),
            pltpu.VMEM((CHG, dh), F32),
            pltpu.VMEM_SHARED((n_tab, dh), F32),
            pltpu.SemaphoreType.DMA,
            pltpu.SemaphoreType.DMA,
            pltpu.SemaphoreType.DMA,
            pltpu.SemaphoreType.DMA,
        ],
    )
    def sc_gather(v_hbm, src_hbm, dst_hbm, gsum_hbm,
                  idx_s, idx_d, rs0, rs1, rd0, rd1, tab,
                  semg0, semg1, semw0, semw1):
        cid = lax.axis_index("c")
        sid = lax.axis_index("s")
        idx_row0 = sid * idx_rows_per_t
        e_base0 = sid * per_t
        col0 = cid * dh
        rows_s = (rs0, rs1)
        rows_d = (rd0, rd1)
        semg = (semg0, semg1)
        semw = (semw0, semw1)
        t0 = sid * tab_rows_per_tile
        pltpu.sync_copy(v_hbm.at[pl.ds(t0, tab_rows_per_tile), pl.ds(col0, dh)],
                        tab.at[pl.ds(t0, tab_rows_per_tile)])
        pltpu.sync_copy(src_hbm.at[pl.ds(idx_row0, idx_rows_per_t)], idx_s)
        pltpu.sync_copy(dst_hbm.at[pl.ds(idx_row0, idx_rows_per_t)], idx_d)
        plsc.subcore_barrier()

        def step(c, b):
            @pl.when(c >= 2)
            def _():
                pltpu.make_async_copy(
                    gsum_hbm.at[0].at[pl.ds(0, CHG)],
                    rows_s[b], semw[b]).wait()
            cg1 = pltpu.async_copy(tab.at[idx_s.at[c]], rows_s[b], semg[b])
            cg2 = pltpu.async_copy(tab.at[idx_d.at[c]], rows_d[b], semg[b])
            cg1.wait()
            cg2.wait()

            def addrow(r, carry):
                for k in range(dh // 16):
                    sl = pl.ds(k * 16, 16)
                    rows_s[b][r, sl] = rows_s[b][r, sl] + rows_d[b][r, sl]
                return carry

            lax.fori_loop(0, CHG, addrow, 0)
            pltpu.async_copy(
                rows_s[b],
                gsum_hbm.at[cid].at[pl.ds(e_base0 + c * CHG, CHG)],
                semw[b])

        def body(i, carry):
            step(2 * i, 0)
            step(2 * i + 1, 1)
            return carry

        lax.fori_loop(0, chunks // 2, body, 0)
        for b in range(2):
            pltpu.make_async_copy(gsum_hbm.at[0].at[pl.ds(0, CHG)],
                                  rows_s[b], semw[b]).wait()

    return sc_gather


def _make_sc_scatter(ep, n_pad):
    per_w = ep // NW
    chunks = per_w // CH
    idx_rows_per_w = per_w // 128
    rows_per_tile = n_pad // NS
    mesh = plsc.VectorSubcoreMesh(core_axis_name="c", subcore_axis_name="s")

    @functools.partial(
        pl.kernel,
        mesh=mesh,
        compiler_params=pltpu.CompilerParams(use_tc_tiling_on_sc=False),
        out_type=jax.ShapeDtypeStruct((NC, n_pad, L), F32),
        scratch_types=[
            pltpu.VMEM((idx_rows_per_w, 128), jnp.int32),
            pltpu.VMEM((CH, L), F32),
            pltpu.VMEM((CH, L), F32),
            pltpu.VMEM_SHARED((n_pad, L), F32),
            pltpu.SemaphoreType.DMA,
            pltpu.SemaphoreType.DMA,
            pltpu.SemaphoreType.DMA,
        ],
    )
    def sc_scatter(h_hbm, dst_hbm, zeros_hbm, s_hbm, idx_d, rows0, rows1,
                   acc, seml0, seml1, sems):
        cid = lax.axis_index("c")
        sid = lax.axis_index("s")
        wid = sid * NC + cid
        tile_r0 = sid * rows_per_tile
        rows = (rows0, rows1)
        seml = (seml0, seml1)
        pltpu.sync_copy(dst_hbm.at[pl.ds(wid * idx_rows_per_w, idx_rows_per_w)],
                        idx_d)
        pltpu.sync_copy(zeros_hbm.at[pl.ds(tile_r0, rows_per_tile)],
                        acc.at[pl.ds(tile_r0, rows_per_tile)])
        plsc.subcore_barrier()
        pltpu.async_copy(h_hbm.at[pl.ds(wid * per_w, CH)], rows[0], seml[0])

        def step(c, b):
            pltpu.make_async_copy(h_hbm.at[pl.ds(0, CH)], rows[b],
                                  seml[b]).wait()

            @pl.when(c + 1 < chunks)
            def _():
                pltpu.async_copy(
                    h_hbm.at[pl.ds(wid * per_w + (c + 1) * CH, CH)],
                    rows[1 - b], seml[1 - b])

            for j in range(KB):
                pltpu.sync_copy(rows[b].at[pl.ds(j * 128, 128)],
                                acc.at[idx_d.at[c * KB + j]], add=True)

        def body(i, carry):
            step(2 * i, 0)
            step(2 * i + 1, 1)
            return carry

        lax.fori_loop(0, chunks // 2, body, 0)
        plsc.subcore_barrier()
        pltpu.sync_copy(acc.at[pl.ds(tile_r0, rows_per_tile)],
                        s_hbm.at[cid].at[pl.ds(tile_r0, rows_per_tile)])

    return sc_scatter


def _make_sc_deg(ep, n_pad):
    per_w = ep // NW
    chunks = per_w // CH
    idx_rows_per_w = per_w // 128
    rows_per_tile = n_pad // NS
    mesh = plsc.VectorSubcoreMesh(core_axis_name="c", subcore_axis_name="s")

    @functools.partial(
        pl.kernel,
        mesh=mesh,
        compiler_params=pltpu.CompilerParams(use_tc_tiling_on_sc=False),
        out_type=jax.ShapeDtypeStruct((NC, n_pad, L), F32),
        scratch_types=[
            pltpu.VMEM((KB, 128), jnp.int32),
            pltpu.VMEM((128, L), F32),
            pltpu.VMEM_SHARED((n_pad, L), F32),
            pltpu.SemaphoreType.DMA,
        ],
    )
    def sc_deg(dst_hbm, zeros_hbm, ones_hbm, s_hbm, idx_d, ones_v, acc, sem):
        cid = lax.axis_index("c")
        sid = lax.axis_index("s")
        wid = sid * NC + cid
        tile_r0 = sid * rows_per_tile
        pltpu.sync_copy(ones_hbm, ones_v)
        pltpu.sync_copy(zeros_hbm.at[pl.ds(tile_r0, rows_per_tile)],
                        acc.at[pl.ds(tile_r0, rows_per_tile)])
        plsc.subcore_barrier()

        def body(ch, carry):
            pltpu.sync_copy(dst_hbm.at[pl.ds(wid * idx_rows_per_w + ch * KB, KB)],
                            idx_d)
            for j in range(KB):
                pltpu.sync_copy(ones_v, acc.at[idx_d.at[j]], add=True)
            return carry

        lax.fori_loop(0, chunks, body, 0)
        plsc.subcore_barrier()
        pltpu.sync_copy(acc.at[pl.ds(tile_r0, rows_per_tile)],
                        s_hbm.at[cid].at[pl.ds(tile_r0, rows_per_tile)])

    return sc_deg


def _bd(w):
    return jnp.kron(jnp.eye(8, dtype=w.dtype), w)


def _row(b):
    return b.reshape(1, -1)


def _tile8(b):
    return jnp.tile(b, 8).reshape(1, -1)


def kernel(x, edge_index, edge_attr, node_enc, edge_enc, mp_mlps, bn_gamma, bn_beta, dec):
    n, d_in = x.shape
    e_num = edge_index.shape[1]
    step = NW * CH
    ep = ((e_num + step - 1) // step) * step
    pad_e = ep - e_num
    n_pad = ((n + 1 + NS * 8 - 1) // (NS * 8)) * (NS * 8)

    src = edge_index[0]
    dst = edge_index[1]
    src_r = jnp.pad(src, (0, pad_e)).reshape(ep // 128, 128)
    dst_gr = jnp.pad(dst, (0, pad_e)).reshape(ep // 128, 128)
    dst_sr = jnp.pad(dst, (0, pad_e), constant_values=n).reshape(ep // 128, 128)
    ea_pack = jnp.pad(edge_attr, ((0, pad_e), (0, 0))).reshape(ep // 8, 8 * L)
    zeros_np = jnp.zeros((n_pad, L), F32)
    ones128 = jnp.ones((128, L), F32)

    (nw1, nb1), (nw2, nb2), (nw3, nb3), (nw4, nb4) = node_enc
    (ew1, eb1), (ew2, eb2), (ew3, eb3), (ew4, eb4) = edge_enc
    (dw1, db1), (dw2, db2), (dw3, db3), (dw4, db4) = dec

    sc_gather = _make_sc_gather(ep, n, 8 * L)
    sc_scatter = _make_sc_scatter(ep, n_pad)
    sc_deg = _make_sc_deg(ep, n_pad)

    v0 = pl.pallas_call(
        _mlp4_body,
        out_shape=jax.ShapeDtypeStruct((n, 8 * L), F32),
    )(x, nw1.T, _row(nb1), nw2.T, _row(nb2), nw3.T, _row(nb3), nw4.T, _row(nb4))

    ep8 = ep // 8
    blk8 = 4096
    grid8 = ep8 // blk8
    dspec8 = pl.BlockSpec((blk8, 8 * L), lambda i: (i, 0))
    wspec = pl.BlockSpec((8 * L, 8 * L), lambda i: (0, 0))
    bspec = pl.BlockSpec((1, 8 * L), lambda i: (0, 0))
    g3_pack = pl.pallas_call(
        _edge_enc_body,
        grid=(grid8,),
        in_specs=[dspec8, wspec, bspec, wspec, bspec, wspec, bspec],
        out_specs=dspec8,
        out_shape=jax.ShapeDtypeStruct((ep8, 8 * L), F32),
    )(ea_pack, _bd(ew1.T), _tile8(eb1), _bd(ew2.T), _tile8(eb2),
      _bd(ew3.T), _tile8(eb3))

    blk_e = 4096
    grid_e = ep // blk_e
    dspec16 = pl.BlockSpec((blk_e, L), lambda i: (i, 0))
    dspec128 = pl.BlockSpec((blk_e, 8 * L), lambda i: (i, 0))
    w16spec = pl.BlockSpec((L, 8 * L), lambda i: (0, 0))
    e_feat = pl.pallas_call(
        _edge_enc4_body,
        grid=(grid_e,),
        in_specs=[dspec16, w16spec, bspec],
        out_specs=dspec128,
        out_shape=jax.ShapeDtypeStruct((ep, 8 * L), F32),
    )(g3_pack.reshape(ep, L), ew4.T, _row(eb4))

    d_pair = sc_deg(dst_sr, zeros_np, ones128)

    vbn = None
    s_pair = None
    for r, p in enumerate(mp_mlps):
        (w1, b1), (w2, b2), (w3, b3), (w4, b4) = p
        if r == 0:
            vbn = pl.pallas_call(
                _bn0_body,
                out_shape=jax.ShapeDtypeStruct((n, 8 * L), F32),
            )(v0, _row(bn_gamma), _row(bn_beta))
        else:
            (pw4, pb4) = mp_mlps[r - 1][3]
            vbn = pl.pallas_call(
                functools.partial(_upd_bn_body, n),
                out_shape=jax.ShapeDtypeStruct((n, 8 * L), F32),
            )(vbn, s_pair, d_pair, pw4.T, _row(pb4),
              _row(bn_gamma), _row(bn_beta))

        gsum = sc_gather(vbn, src_r, dst_gr)

        h1 = pl.pallas_call(
            _mp_l1_body,
            grid=(grid_e,),
            in_specs=[pl.BlockSpec((1, blk_e, 4 * L), lambda i: (0, i, 0)),
                      pl.BlockSpec((1, blk_e, 4 * L), lambda i: (1, i, 0)),
                      dspec128,
                      pl.BlockSpec((8 * L, L), lambda i: (0, 0)),
                      pl.BlockSpec((1, L), lambda i: (0, 0))],
            out_specs=dspec16,
            out_shape=jax.ShapeDtypeStruct((ep, L), F32),
        )(gsum, gsum, e_feat, w1.T, _row(b1))

        q3_pack = pl.pallas_call(
            _mp_l23_body,
            grid=(grid8,),
            in_specs=[dspec8, wspec, bspec, wspec, bspec],
            out_specs=dspec8,
            out_shape=jax.ShapeDtypeStruct((ep8, 8 * L), F32),
        )(h1.reshape(ep8, 8 * L), _bd(w2.T), _tile8(b2), _bd(w3.T), _tile8(b3))

        s_pair = sc_scatter(q3_pack.reshape(ep, L), dst_sr, zeros_np)

    (lw4, lb4) = mp_mlps[-1][3]
    out = pl.pallas_call(
        functools.partial(_upd_dec_body, n),
        out_shape=jax.ShapeDtypeStruct((n, 8 * L), F32),
    )(vbn, s_pair, d_pair, lw4.T, _row(lb4),
      dw1.T, _row(db1), dw2.T, _row(db2), dw3.T, _row(db3), dw4.T, _row(db4))
    return out


# speedup vs baseline: 1.4372x; 1.4372x over previous
import functools

import jax
import jax.numpy as jnp
from jax import lax
from jax.experimental import pallas as pl
from jax.experimental.pallas import tpu as pltpu
from jax.experimental.pallas import tpu_sc as plsc

F32 = jnp.float32
BF16 = jnp.bfloat16
L = 16
NC = 2
NS = 16
NW = NC * NS
CH = 1024
KB = CH // 128
CHG = 128


def _gelu(x):
    return x * 0.5 * (1.0 + lax.erf(x * 0.7071067811865476))


def _qdot(a, b):
    return jnp.dot(a.astype(BF16), b.astype(BF16), preferred_element_type=F32)


def _fdot(a, b):
    return jnp.dot(a, b, preferred_element_type=F32,
                   precision=lax.Precision.HIGHEST)


def _mlp4_body(x_ref, w1, b1, w2, b2, w3, b3, w4, b4, o_ref):
    h = _gelu(_qdot(x_ref[:], w1[:]) + b1[:])
    h = _gelu(_qdot(h, w2[:]) + b2[:])
    h = _gelu(_qdot(h, w3[:]) + b3[:])
    o_ref[:] = _qdot(h, w4[:]) + b4[:]


def _bn_body(v, g_ref, bta_ref):
    mean = jnp.mean(v, axis=0, keepdims=True)
    var = jnp.mean((v - mean) ** 2, axis=0, keepdims=True)
    return (v - mean) / jnp.sqrt(var + 1e-5) * g_ref[:] + bta_ref[:]


def _bn0_body(v_ref, g_ref, bta_ref, vbn_ref):
    vbn_ref[:] = _bn_body(v_ref[:], g_ref, bta_ref)


def _update(n_rows, v_ref, s_ref, d_ref, w4t_ref, b4_ref):
    s = s_ref[0, :n_rows, :] + s_ref[1, :n_rows, :]
    deg = d_ref[0, :n_rows, 0:1] + d_ref[1, :n_rows, 0:1]
    w4q = w4t_ref[:].astype(BF16).astype(F32)
    return v_ref[:] + _fdot(s, w4q) + deg * b4_ref[:]


def _upd_bn_body(n_rows, v_ref, s_ref, d_ref, w4t_ref, b4_ref,
                 g_ref, bta_ref, vbn_ref):
    vbn_ref[:] = _bn_body(_update(n_rows, v_ref, s_ref, d_ref, w4t_ref, b4_ref),
                          g_ref, bta_ref)


def _upd_dec_body(n_rows, v_ref, s_ref, d_ref, w4t_ref, b4_ref,
                  dw1, db1, dw2, db2, dw3, db3, dw4, db4, o_ref):
    v = _update(n_rows, v_ref, s_ref, d_ref, w4t_ref, b4_ref)
    h = _gelu(_qdot(v, dw1[:]) + db1[:])
    h = _gelu(_qdot(h, dw2[:]) + db2[:])
    h = _gelu(_qdot(h, dw3[:]) + db3[:])
    o_ref[:] = _qdot(h, dw4[:]) + db4[:]


def _edge_enc_body(ea_ref, w1, b1, w2, b2, w3, b3, o_ref):
    h = _gelu(_qdot(ea_ref[:], w1[:]) + b1[:])
    h = _gelu(_qdot(h, w2[:]) + b2[:])
    o_ref[:] = _gelu(_qdot(h, w3[:]) + b3[:])


def _edge_enc4_body(g3_ref, w4, b4, o_ref):
    o_ref[:] = _qdot(g3_ref[:], w4[:]) + b4[:]


def _mp_l1_body(gsum_ref, e_ref, w1, b1, o_ref):
    m = gsum_ref[:] + e_ref[:]
    o_ref[:] = _gelu(_qdot(m, w1[:]) + b1[:])


def _mp_l23_body(h_ref, w2, b2, w3, b3, o_ref):
    h = _gelu(_qdot(h_ref[:], w2[:]) + b2[:])
    h = _gelu(_qdot(h, w3[:]) + b3[:])
    o_ref[:] = h.astype(BF16).astype(F32)


def _make_sc_gather(ep, n_tab, d):
    dh = d // 2
    per_t = ep // NS
    chunks = per_t // CHG
    idx_rows_per_t = per_t // 128
    tab_rows_per_tile = n_tab // NS
    mesh = plsc.VectorSubcoreMesh(core_axis_name="c", subcore_axis_name="s")

    @functools.partial(
        pl.kernel,
        mesh=mesh,
        compiler_params=pltpu.CompilerParams(use_tc_tiling_on_sc=False),
        out_type=jax.ShapeDtypeStruct((ep, d), F32),
        scratch_types=[
            pltpu.VMEM((idx_rows_per_t, 128), jnp.int32),
            pltpu.VMEM((idx_rows_per_t, 128), jnp.int32),
            pltpu.VMEM((CHG, dh), F32),
            pltpu.VMEM((CHG, dh), F32),
            pltpu.VMEM((CHG, dh), F32),
            pltpu.VMEM((CHG, dh), F32),
            pltpu.VMEM_SHARED((n_tab, dh), F32),
            pltpu.SemaphoreType.DMA,
            pltpu.SemaphoreType.DMA,
            pltpu.SemaphoreType.DMA,
            pltpu.SemaphoreType.DMA,
        ],
    )
    def sc_gather(v_hbm, src_hbm, dst_hbm, gsum_hbm,
                  idx_s, idx_d, rs0, rs1, rd0, rd1, tab,
                  semg0, semg1, semw0, semw1):
        cid = lax.axis_index("c")
        sid = lax.axis_index("s")
        idx_row0 = sid * idx_rows_per_t
        e_base0 = sid * per_t
        col0 = cid * dh
        rows_s = (rs0, rs1)
        rows_d = (rd0, rd1)
        semg = (semg0, semg1)
        semw = (semw0, semw1)
        t0 = sid * tab_rows_per_tile
        pltpu.sync_copy(v_hbm.at[pl.ds(t0, tab_rows_per_tile), pl.ds(col0, dh)],
                        tab.at[pl.ds(t0, tab_rows_per_tile)])
        pltpu.sync_copy(src_hbm.at[pl.ds(idx_row0, idx_rows_per_t)], idx_s)
        pltpu.sync_copy(dst_hbm.at[pl.ds(idx_row0, idx_rows_per_t)], idx_d)
        plsc.subcore_barrier()

        def step(c, b):
            @pl.when(c >= 2)
            def _():
                pltpu.make_async_copy(
                    gsum_hbm.at[pl.ds(0, CHG), pl.ds(col0, dh)],
                    rows_s[b], semw[b]).wait()
            cg1 = pltpu.async_copy(tab.at[idx_s.at[c]], rows_s[b], semg[b])
            cg2 = pltpu.async_copy(tab.at[idx_d.at[c]], rows_d[b], semg[b])
            cg1.wait()
            cg2.wait()

            def addrow(r, carry):
                for k in range(dh // 16):
                    sl = pl.ds(k * 16, 16)
                    rows_s[b][r, sl] = rows_s[b][r, sl] + rows_d[b][r, sl]
                return carry

            lax.fori_loop(0, CHG, addrow, 0)
            pltpu.async_copy(
                rows_s[b],
                gsum_hbm.at[pl.ds(e_base0 + c * CHG, CHG), pl.ds(col0, dh)],
                semw[b])

        def body(i, carry):
            step(2 * i, 0)
            step(2 * i + 1, 1)
            return carry

        lax.fori_loop(0, chunks // 2, body, 0)
        for b in range(2):
            pltpu.make_async_copy(gsum_hbm.at[pl.ds(0, CHG), pl.ds(col0, dh)],
                                  rows_s[b], semw[b]).wait()

    return sc_gather


def _make_sc_scatter(ep, n_pad):
    per_w = ep // NW
    chunks = per_w // CH
    idx_rows_per_w = per_w // 128
    rows_per_tile = n_pad // NS
    mesh = plsc.VectorSubcoreMesh(core_axis_name="c", subcore_axis_name="s")

    @functools.partial(
        pl.kernel,
        mesh=mesh,
        compiler_params=pltpu.CompilerParams(use_tc_tiling_on_sc=False),
        out_type=jax.ShapeDtypeStruct((NC, n_pad, L), F32),
        scratch_types=[
            pltpu.VMEM((idx_rows_per_w, 128), jnp.int32),
            pltpu.VMEM((CH, L), F32),
            pltpu.VMEM((CH, L), F32),
            pltpu.VMEM_SHARED((n_pad, L), F32),
            pltpu.SemaphoreType.DMA,
            pltpu.SemaphoreType.DMA,
            pltpu.SemaphoreType.DMA,
        ],
    )
    def sc_scatter(h_hbm, dst_hbm, zeros_hbm, s_hbm, idx_d, rows0, rows1,
                   acc, seml0, seml1, sems):
        cid = lax.axis_index("c")
        sid = lax.axis_index("s")
        wid = sid * NC + cid
        tile_r0 = sid * rows_per_tile
        rows = (rows0, rows1)
        seml = (seml0, seml1)
        pltpu.sync_copy(dst_hbm.at[pl.ds(wid * idx_rows_per_w, idx_rows_per_w)],
                        idx_d)
        pltpu.sync_copy(zeros_hbm.at[pl.ds(tile_r0, rows_per_tile)],
                        acc.at[pl.ds(tile_r0, rows_per_tile)])
        plsc.subcore_barrier()
        pltpu.async_copy(h_hbm.at[pl.ds(wid * per_w, CH)], rows[0], seml[0])

        def step(c, b):
            pltpu.make_async_copy(h_hbm.at[pl.ds(0, CH)], rows[b],
                                  seml[b]).wait()

            @pl.when(c + 1 < chunks)
            def _():
                pltpu.async_copy(
                    h_hbm.at[pl.ds(wid * per_w + (c + 1) * CH, CH)],
                    rows[1 - b], seml[1 - b])

            for j in range(KB):
                pltpu.sync_copy(rows[b].at[pl.ds(j * 128, 128)],
                                acc.at[idx_d.at[c * KB + j]], add=True)

        def body(i, carry):
            step(2 * i, 0)
            step(2 * i + 1, 1)
            return carry

        lax.fori_loop(0, chunks // 2, body, 0)
        plsc.subcore_barrier()
        pltpu.sync_copy(acc.at[pl.ds(tile_r0, rows_per_tile)],
                        s_hbm.at[cid].at[pl.ds(tile_r0, rows_per_tile)])

    return sc_scatter


def _make_sc_deg(ep, n_pad):
    per_w = ep // NW
    chunks = per_w // CH
    idx_rows_per_w = per_w // 128
    rows_per_tile = n_pad // NS
    mesh = plsc.VectorSubcoreMesh(core_axis_name="c", subcore_axis_name="s")

    @functools.partial(
        pl.kernel,
        mesh=mesh,
        compiler_params=pltpu.CompilerParams(use_tc_tiling_on_sc=False),
        out_type=jax.ShapeDtypeStruct((NC, n_pad, L), F32),
        scratch_types=[
            pltpu.VMEM((KB, 128), jnp.int32),
            pltpu.VMEM((128, L), F32),
            pltpu.VMEM_SHARED((n_pad, L), F32),
            pltpu.SemaphoreType.DMA,
        ],
    )
    def sc_deg(dst_hbm, zeros_hbm, ones_hbm, s_hbm, idx_d, ones_v, acc, sem):
        cid = lax.axis_index("c")
        sid = lax.axis_index("s")
        wid = sid * NC + cid
        tile_r0 = sid * rows_per_tile
        pltpu.sync_copy(ones_hbm, ones_v)
        pltpu.sync_copy(zeros_hbm.at[pl.ds(tile_r0, rows_per_tile)],
                        acc.at[pl.ds(tile_r0, rows_per_tile)])
        plsc.subcore_barrier()

        def body(ch, carry):
            pltpu.sync_copy(dst_hbm.at[pl.ds(wid * idx_rows_per_w + ch * KB, KB)],
                            idx_d)
            for j in range(KB):
                pltpu.sync_copy(ones_v, acc.at[idx_d.at[j]], add=True)
            return carry

        lax.fori_loop(0, chunks, body, 0)
        plsc.subcore_barrier()
        pltpu.sync_copy(acc.at[pl.ds(tile_r0, rows_per_tile)],
                        s_hbm.at[cid].at[pl.ds(tile_r0, rows_per_tile)])

    return sc_deg


def _bd(w):
    return jnp.kron(jnp.eye(8, dtype=w.dtype), w)


def _row(b):
    return b.reshape(1, -1)


def _tile8(b):
    return jnp.tile(b, 8).reshape(1, -1)


def kernel(x, edge_index, edge_attr, node_enc, edge_enc, mp_mlps, bn_gamma, bn_beta, dec):
    n, d_in = x.shape
    e_num = edge_index.shape[1]
    step = NW * CH
    ep = ((e_num + step - 1) // step) * step
    pad_e = ep - e_num
    n_pad = ((n + 1 + NS * 8 - 1) // (NS * 8)) * (NS * 8)

    src = edge_index[0]
    dst = edge_index[1]
    src_r = jnp.pad(src, (0, pad_e)).reshape(ep // 128, 128)
    dst_gr = jnp.pad(dst, (0, pad_e)).reshape(ep // 128, 128)
    dst_sr = jnp.pad(dst, (0, pad_e), constant_values=n).reshape(ep // 128, 128)
    ea_pack = jnp.pad(edge_attr, ((0, pad_e), (0, 0))).reshape(ep // 8, 8 * L)
    zeros_np = jnp.zeros((n_pad, L), F32)
    ones128 = jnp.ones((128, L), F32)

    (nw1, nb1), (nw2, nb2), (nw3, nb3), (nw4, nb4) = node_enc
    (ew1, eb1), (ew2, eb2), (ew3, eb3), (ew4, eb4) = edge_enc
    (dw1, db1), (dw2, db2), (dw3, db3), (dw4, db4) = dec

    sc_gather = _make_sc_gather(ep, n, 8 * L)
    sc_scatter = _make_sc_scatter(ep, n_pad)
    sc_deg = _make_sc_deg(ep, n_pad)

    v0 = pl.pallas_call(
        _mlp4_body,
        out_shape=jax.ShapeDtypeStruct((n, 8 * L), F32),
    )(x, nw1.T, _row(nb1), nw2.T, _row(nb2), nw3.T, _row(nb3), nw4.T, _row(nb4))

    ep8 = ep // 8
    blk8 = 4096
    grid8 = ep8 // blk8
    dspec8 = pl.BlockSpec((blk8, 8 * L), lambda i: (i, 0))
    wspec = pl.BlockSpec((8 * L, 8 * L), lambda i: (0, 0))
    bspec = pl.BlockSpec((1, 8 * L), lambda i: (0, 0))
    g3_pack = pl.pallas_call(
        _edge_enc_body,
        grid=(grid8,),
        in_specs=[dspec8, wspec, bspec, wspec, bspec, wspec, bspec],
        out_specs=dspec8,
        out_shape=jax.ShapeDtypeStruct((ep8, 8 * L), F32),
    )(ea_pack, _bd(ew1.T), _tile8(eb1), _bd(ew2.T), _tile8(eb2),
      _bd(ew3.T), _tile8(eb3))

    blk_e = 4096
    grid_e = ep // blk_e
    dspec16 = pl.BlockSpec((blk_e, L), lambda i: (i, 0))
    dspec128 = pl.BlockSpec((blk_e, 8 * L), lambda i: (i, 0))
    w16spec = pl.BlockSpec((L, 8 * L), lambda i: (0, 0))
    e_feat = pl.pallas_call(
        _edge_enc4_body,
        grid=(grid_e,),
        in_specs=[dspec16, w16spec, bspec],
        out_specs=dspec128,
        out_shape=jax.ShapeDtypeStruct((ep, 8 * L), F32),
    )(g3_pack.reshape(ep, L), ew4.T, _row(eb4))

    d_pair = sc_deg(dst_sr, zeros_np, ones128)

    vbn = None
    s_pair = None
    for r, p in enumerate(mp_mlps):
        (w1, b1), (w2, b2), (w3, b3), (w4, b4) = p
        if r == 0:
            vbn = pl.pallas_call(
                _bn0_body,
                out_shape=jax.ShapeDtypeStruct((n, 8 * L), F32),
            )(v0, _row(bn_gamma), _row(bn_beta))
        else:
            (pw4, pb4) = mp_mlps[r - 1][3]
            vbn = pl.pallas_call(
                functools.partial(_upd_bn_body, n),
                out_shape=jax.ShapeDtypeStruct((n, 8 * L), F32),
            )(vbn, s_pair, d_pair, pw4.T, _row(pb4),
              _row(bn_gamma), _row(bn_beta))

        gsum = sc_gather(vbn, src_r, dst_gr)

        h1 = pl.pallas_call(
            _mp_l1_body,
            grid=(grid_e,),
            in_specs=[dspec128, dspec128,
                      pl.BlockSpec((8 * L, L), lambda i: (0, 0)),
                      pl.BlockSpec((1, L), lambda i: (0, 0))],
            out_specs=dspec16,
            out_shape=jax.ShapeDtypeStruct((ep, L), F32),
        )(gsum, e_feat, w1.T, _row(b1))

        q3_pack = pl.pallas_call(
            _mp_l23_body,
            grid=(grid8,),
            in_specs=[dspec8, wspec, bspec, wspec, bspec],
            out_specs=dspec8,
            out_shape=jax.ShapeDtypeStruct((ep8, 8 * L), F32),
        )(h1.reshape(ep8, 8 * L), _bd(w2.T), _tile8(b2), _bd(w3.T), _tile8(b3))

        s_pair = sc_scatter(q3_pack.reshape(ep, L), dst_sr, zeros_np)

    (lw4, lb4) = mp_mlps[-1][3]
    out = pl.pallas_call(
        functools.partial(_upd_dec_body, n),
        out_shape=jax.ShapeDtypeStruct((n, 8 * L), F32),
    )(vbn, s_pair, d_pair, lw4.T, _row(lb4),
      dw1.T, _row(db1), dw2.T, _row(db2), dw3.T, _row(db3), dw4.T, _row(db4))
    return out
